# Initial kernel scaffold; baseline (speedup 1.0000x reference)
import jax
import jax.numpy as jnp
from jax.experimental import pallas as pl


def kernel(X, Y, Pxy, Pyx, thetas):
    raise NotImplementedError("write your pallas kernel here")


# speedup vs baseline: 10203.9006x; 10203.9006x over previous
import jax
import jax.numpy as jnp
from jax import lax
from jax.experimental import pallas as pl
from jax.experimental.pallas import tpu as pltpu

_N = 2048
_D = 128
_L = 100
_LP = 128


def _roll_up(x, j):
    return jnp.concatenate([x[j:], x[:j]], axis=0)


def _roll_down(x, j):
    return jnp.concatenate([x[-j:], x[:-j]], axis=0)


def _sort_ssd_kernel(x_ref, y_ref, th_ref, out_ref):
    th = th_ref[0]
    norm2 = jnp.sum(th * th, axis=1, keepdims=True)
    tn = th / jnp.sqrt(norm2)

    dn = (((1,), (1,)), ((), ()))
    x = x_ref[0]
    y = y_ref[0]
    px = lax.dot_general(x, tn, dn, preferred_element_type=jnp.float32)
    py = lax.dot_general(y, tn, dn, preferred_element_type=jnp.float32)

    s = jnp.concatenate([px, py], axis=1)
    row = lax.broadcasted_iota(jnp.int32, (_N, 1), 0)

    k = 2
    while k <= _N:
        j = k // 2
        while j >= 1:
            up = (row & k) == 0
            low = (row & j) == 0
            part = jnp.where(low, _roll_up(s, j), _roll_down(s, j))
            mn = jnp.minimum(s, part)
            mx = jnp.maximum(s, part)
            s = jnp.where(up == low, mn, mx)
            j //= 2
        k *= 2

    d = s[:, :_LP] - s[:, _LP:]
    ssd = jnp.sum(d * d, axis=0, keepdims=True)
    col = lax.broadcasted_iota(jnp.int32, (1, _LP), 1)
    out_ref[0] = jnp.where(col < _L, ssd, 0.0)


def _final_kernel(ssd_ref, out_ref):
    v = ssd_ref[...]
    tot = jnp.sum(v, axis=1, keepdims=True)
    sw = jnp.sqrt(tot / (_N * _L))
    out_ref[0, 0] = jnp.sum(sw) / v.shape[0]


def kernel(X, Y, Pxy, Pyx, thetas):
    B = X.shape[0]
    th_p = jnp.pad(thetas, ((0, 0), (0, _LP - _L), (0, 0)))

    ssd = pl.pallas_call(
        _sort_ssd_kernel,
        grid=(B,),
        in_specs=[
            pl.BlockSpec((1, _N, _D), lambda b: (b, 0, 0)),
            pl.BlockSpec((1, _N, _D), lambda b: (b, 0, 0)),
            pl.BlockSpec((1, _LP, _D), lambda b: (b, 0, 0)),
        ],
        out_specs=pl.BlockSpec((1, 1, _LP), lambda b: (b, 0, 0)),
        out_shape=jax.ShapeDtypeStruct((B, 1, _LP), jnp.float32),
    )(X, Y, th_p)

    out = pl.pallas_call(
        _final_kernel,
        in_specs=[pl.BlockSpec((B, _LP), lambda: (0, 0))],
        out_specs=pl.BlockSpec(memory_space=pltpu.SMEM),
        out_shape=jax.ShapeDtypeStruct((1, 1), jnp.float32),
    )(ssd.reshape(B, _LP))

    return out[0, 0]
